# baseline (speedup 1.0000x reference)
import jax
import jax.numpy as jnp
from jax.experimental import pallas as pl


def _conv(x, w, b, stride, pad):
    y = jax.lax.conv_general_dilated(
        x, w, (stride, stride), [(pad, pad), (pad, pad)],
        dimension_numbers=('NCHW', 'OIHW', 'NCHW'))
    return y + b[None, :, None, None]


def _conv1_s2d(img, W1, b1):
    xp = jnp.pad(img, ((0, 0), (0, 0), (3, 5), (3, 5)))
    xs = jnp.stack([xp[:, :, 0::2, 0::2], xp[:, :, 0::2, 1::2],
                    xp[:, :, 1::2, 0::2], xp[:, :, 1::2, 1::2]],
                   axis=2).reshape(8, 12, 116, 116)
    wp = jnp.pad(W1, ((0, 0), (0, 0), (0, 1), (0, 1)))
    wr = wp.reshape(64, 3, 4, 2, 4, 2)
    w2 = wr.transpose(0, 1, 3, 5, 2, 4).reshape(64, 12, 4, 4)
    y = jax.lax.conv_general_dilated(
        xs, w2, (1, 1), [(0, 0), (0, 0)],
        dimension_numbers=('NCHW', 'OIHW', 'NCHW'))[:, :, :112, :112]
    return y + b1[None, :, None, None]


def _body(xs_ref, out_ref):
    xs = xs_ref[...]
    s = jnp.sum(xs)
    out_ref[...] = jnp.full((8, 256, 4), s, jnp.float32)


def kernel(img, W1, b1, W2, b2, Wc, bc):
    x = jax.nn.relu(_conv1_s2d(img, W1, b1))
    x = jax.nn.relu(_conv(x, W2, b2, 2, 1))
    x = _conv(x, Wc, bc, 1, 0)
    x = jax.nn.sigmoid(x)
    return pl.pallas_call(
        _body,
        out_shape=jax.ShapeDtypeStruct((8, 256, 4), jnp.float32),
    )(x.reshape(448, 56))


# speedup vs baseline: 1.4805x; 1.4805x over previous
import jax
import jax.numpy as jnp
from jax.experimental import pallas as pl
from jax.experimental.pallas import tpu as pltpu

HP = jax.lax.Precision.HIGHEST
B = 8
R = 56
C = 56
G = B * R
K = 256
KK = B * K
IH = 224
GI = B * IH


def _conv(x, w, b, stride, pad):
    y = jax.lax.conv_general_dilated(
        x, w, (stride, stride), [(pad, pad), (pad, pad)],
        dimension_numbers=('NCHW', 'OIHW', 'NCHW'))
    return y + b[None, :, None, None]


def _iota(shape, dim):
    return jax.lax.broadcasted_iota(jnp.int32, shape, dim)


def _body(xs_ref, img_ref, out_ref):
    f32 = jnp.float32
    xs = xs_ref[...]
    rmin = jnp.min(xs, axis=-1, keepdims=True)
    rmax = jnp.max(xs, axis=-1, keepdims=True)
    xn = (xs - rmin) / (rmax - rmin)

    a4 = img_ref[...]
    a = jnp.maximum(jnp.maximum(a4[0], a4[1]),
                    jnp.maximum(a4[2], a4[3]))
    wl = jnp.maximum(jnp.maximum(a, jnp.roll(a, -1, axis=1)),
                     jnp.maximum(jnp.roll(a, -2, axis=1),
                                 jnp.roll(a, -3, axis=1)))
    selL = jnp.where(_iota((IH, C), 0) == 4 * _iota((IH, C), 1), 1.0, 0.0)
    imgp = jax.lax.dot(wl, selL, precision=HP)
    imin = jnp.min(imgp, axis=-1, keepdims=True)
    h = xn * (imgp - imin)

    bits3 = jax.lax.bitcast_convert_type(h, jnp.int32).reshape(B, R, C)
    t = jnp.zeros((B, 1, 1), jnp.int32)
    for bit in range(29, -1, -1):
        tc = t + (1 << bit)
        ge = jnp.sum(jnp.where(bits3 >= tc, 1, 0),
                     axis=2, keepdims=True).sum(axis=1, keepdims=True)
        t = jnp.where(ge >= K, tc, t)
    m = jnp.sum(jnp.where(bits3 > t, 1, 0),
                axis=2, keepdims=True).sum(axis=1, keepdims=True)
    r_need = (K - m).astype(f32)
    eqf = jnp.where(bits3 == t, 1.0, 0.0).reshape(G, C)
    gtf = jnp.where(bits3 > t, 1.0, 0.0).reshape(G, C)
    rneedR = jnp.broadcast_to(r_need, (B, R, 1)).reshape(G, 1)

    U = jnp.where(_iota((C, C), 0) <= _iota((C, C), 1), 1.0, 0.0)
    samerow = _iota((G, G), 0) // R == _iota((G, G), 1) // R
    Lbd = jnp.where(samerow & (_iota((G, G), 0) > _iota((G, G), 1)), 1.0, 0.0)
    onesC = jnp.ones((C, 1), f32)

    within_eq = jax.lax.dot(eqf, U, precision=HP)
    carry_eq = jax.lax.dot(Lbd, within_eq[:, C - 1:C], precision=HP)
    eq_excl = within_eq + carry_eq - eqf
    sel = gtf + eqf * jnp.where(eq_excl < rneedR, 1.0, 0.0)
    rowpos = jax.lax.dot(sel, U, precision=HP) - sel
    cnt = jax.lax.dot(sel, onesC, precision=HP)
    start = jax.lax.dot(Lbd, cnt, precision=HP)

    rloc = (_iota((G, 1), 0) % R).astype(f32)
    X = jnp.concatenate([sel * (rowpos + start), sel * h, sel, sel * rloc],
                        axis=1)
    XT = jnp.transpose(X)
    I448 = jnp.where(_iota((G, G), 0) == _iota((G, G), 1), 1.0, 0.0)
    startT = jax.lax.dot_general(start, I448, (((0,), (0,)), ((), ())),
                                 precision=HP)
    cntT = jax.lax.dot_general(cnt, I448, (((0,), (0,)), ((), ())),
                               precision=HP)
    kmodC = (_iota((KK, 1), 0) % K).astype(f32)
    ksmpC = _iota((KK, 1), 0) // K
    gsmpT = _iota((1, G), 1) // R
    row1h = jnp.where((ksmpC == gsmpT) & (kmodC >= startT)
                      & (kmodC < startT + cntT), 1.0, 0.0)
    kmodT = (_iota((1, KK), 1) % K).astype(f32)
    ksmpT = _iota((1, KK), 1) // K
    gsmpC = _iota((G, 1), 0) // R
    row1hT = jnp.where((gsmpC == ksmpT) & (kmodT >= start)
                       & (kmodT < start + cnt), 1.0, 0.0)

    big = jax.lax.dot(row1h, X, precision=HP)
    bigT = jax.lax.dot(XT, row1hT, precision=HP)
    mpos, mval = big[:, :C], big[:, C:2 * C]
    msel, mrow = big[:, 2 * C:3 * C], big[:, 3 * C:4 * C]
    mposT, mvalT = bigT[:C], bigT[C:2 * C]
    mselT, mrowT = bigT[2 * C:3 * C], bigT[3 * C:4 * C]

    col1h = msel * jnp.where(mpos == kmodC, 1.0, 0.0)
    col1hT = mselT * jnp.where(mposT == kmodT, 1.0, 0.0)
    ciota = _iota((C, 1), 0).astype(f32)
    ciotaT = _iota((1, C), 1).astype(f32)
    onesT = jnp.ones((1, C), f32)
    val = jax.lax.dot(col1h * mval, onesC, precision=HP)
    valT = jax.lax.dot(onesT, col1hT * mvalT, precision=HP)
    rowidx = jax.lax.dot(col1h * mrow, onesC, precision=HP)
    rowidxT = jax.lax.dot(onesT, col1hT * mrowT, precision=HP)
    colidx = jax.lax.dot(col1h, ciota, precision=HP)
    colidxT = jax.lax.dot(ciotaT, col1hT, precision=HP)
    flat = rowidx * float(C) + colidx
    flatT = rowidxT * float(C) + colidxT

    val3 = val.reshape(B, K, 1)
    flat3 = flat.reshape(B, K, 1)
    valT3 = valT.reshape(B, 1, K)
    flatT3 = flatT.reshape(B, 1, K)
    colT3 = colidxT.reshape(B, 1, K)
    rowT3 = rowidxT.reshape(B, 1, K)
    beats = jnp.where((val3 > valT3)
                      | ((val3 == valT3) & (flat3 < flatT3)), 1.0, 0.0)
    rankL = jnp.sum(beats, axis=1, keepdims=True)
    kio3 = _iota((1, K, 1), 1).astype(f32)
    perm = jnp.where(kio3 == rankL, 1.0, 0.0)
    outc = jnp.sum(perm * colT3, axis=2, keepdims=True)
    outr = jnp.sum(perm * rowT3, axis=2, keepdims=True)
    outv = jnp.sum(perm * valT.reshape(B, 1, K), axis=2, keepdims=True)
    out_ref[...] = jnp.concatenate(
        [outc, outr, jnp.zeros((B, K, 1), f32), outv], axis=2)


def kernel(img, W1, b1, W2, b2, Wc, bc):
    x = jax.nn.relu(_conv(img, W1, b1, 2, 3))
    x = jax.nn.relu(_conv(x, W2, b2, 2, 1))
    x = _conv(x, Wc, bc, 1, 0)
    x = jax.nn.sigmoid(x)
    img4 = img[:, 0].reshape(B, R, 4, IH).transpose(2, 0, 1, 3)
    return pl.pallas_call(
        _body,
        out_shape=jax.ShapeDtypeStruct((B, K, 4), jnp.float32),
    )(x.reshape(G, C), img4.reshape(4, G, IH))
